# baseline (speedup 1.0000x reference)
import jax
import jax.numpy as jnp
from jax import lax
from jax.experimental import pallas as pl
from jax.experimental.pallas import tpu as pltpu
from jax.experimental.pallas import tpu_sc as plsc
from jax.sharding import PartitionSpec as P

_LANES = 16
_NW = 32
_WIN = 10
_BAND = 2 * _WIN + 1


def _dtw_knn_body(a_hbm, fit_hbm, lab_hbm, lab_out, best_out, besti_out,
                  a_v, fit_v, lab_v, res_v, best_v, besti_v):
    t = fit_hbm.shape[0] // lab_v.shape[0]
    nkeys = lab_v.shape[0]
    wid = lax.axis_index("s") * 2 + lax.axis_index("c")
    pltpu.sync_copy(a_hbm.at[wid], a_v)
    pltpu.sync_copy(fit_hbm, fit_v)
    pltpu.sync_copy(lab_hbm, lab_v)

    inf = jnp.full((_LANES,), jnp.inf, jnp.float32)

    def bcast_b(idx):
        return plsc.load_gather(fit_v, [idx])

    def key_body(jkey, carry):
        best, besti = carry
        kidx = jnp.full((_LANES,), jkey, jnp.int32)
        kbase = kidx * t
        klim = kbase + (t - 1)

        a0 = a_v[pl.ds(0, _LANES)]
        st = [inf] * _BAND
        run = jnp.abs(a0 - bcast_b(kbase))
        st[_WIN] = run
        for k in range(_WIN + 1, _BAND):
            run = run + jnp.abs(a0 - bcast_b(kbase + (k - _WIN)))
            st[k] = run

        def make_row(clamp_lo, clamp_hi):
            def row_body(i, carry_t):
                rb = carry_t[0]
                st = list(carry_t[1:])
                ai = a_v[pl.ds(i * _LANES, _LANES)]
                for k in range(_BAND - 1):
                    idx = rb + k if k else rb
                    if clamp_lo and k < _WIN:
                        cidx = jnp.maximum(idx, kbase)
                    elif clamp_hi and k > _WIN:
                        cidx = jnp.minimum(idx, klim)
                    else:
                        cidx = idx
                    c = jnp.abs(ai - bcast_b(cidx))
                    left = st[k - 1] if k >= 1 else inf
                    val = jnp.minimum(jnp.minimum(st[k], st[k + 1]), left) + c
                    if clamp_lo and k < _WIN:
                        val = jnp.where(idx >= kbase, val, inf)
                    elif clamp_hi and k > _WIN:
                        val = jnp.where(idx <= klim, val, inf)
                    st[k] = val
                st[_BAND - 1] = inf
                return (rb + 1,) + tuple(st)
            return row_body

        rb0 = kbase + (1 - _WIN)
        carry_t = (rb0,) + tuple(st)
        carry_t = lax.fori_loop(1, _WIN + 1, make_row(True, False), carry_t)
        carry_t = lax.fori_loop(_WIN + 1, t - _WIN + 1, make_row(False, False),
                                carry_t)
        carry_t = lax.fori_loop(t - _WIN + 1, t, make_row(False, True), carry_t)

        dist = carry_t[1 + _WIN]
        upd = dist < best
        best = jnp.where(upd, dist, best)
        besti = jnp.where(upd, kidx, besti)
        return best, besti

    best, besti = lax.fori_loop(
        0, nkeys,
        key_body,
        (inf, jnp.zeros((_LANES,), jnp.int32)),
    )
    res_v[...] = plsc.load_gather(lab_v, [besti])
    best_v[...] = best
    besti_v[...] = besti
    off = wid * _LANES
    pltpu.sync_copy(res_v, lab_out.at[pl.ds(off, _LANES)])
    pltpu.sync_copy(best_v, best_out.at[pl.ds(off, _LANES)])
    pltpu.sync_copy(besti_v, besti_out.at[pl.ds(off, _LANES)])


def _run_shard(samples, fit_data, fit_labels):
    s, t = samples.shape
    per_w = s // _NW
    a_resh = samples.reshape(_NW, per_w, t).transpose(0, 2, 1).reshape(_NW, -1)
    fit_flat = fit_data.reshape(-1)
    mesh = plsc.VectorSubcoreMesh(core_axis_name="c", subcore_axis_name="s")
    f = pl.kernel(
        _dtw_knn_body,
        out_type=(
            jax.ShapeDtypeStruct((s,), jnp.int32),
            jax.ShapeDtypeStruct((s,), jnp.float32),
            jax.ShapeDtypeStruct((s,), jnp.int32),
        ),
        mesh=mesh,
        compiler_params=pltpu.CompilerParams(needs_layout_passes=False),
        scratch_types=[
            pltpu.VMEM((t * per_w,), jnp.float32),
            pltpu.VMEM((fit_flat.shape[0],), jnp.float32),
            pltpu.VMEM(fit_labels.shape, jnp.int32),
            pltpu.VMEM((per_w,), jnp.int32),
            pltpu.VMEM((per_w,), jnp.float32),
            pltpu.VMEM((per_w,), jnp.int32),
        ],
    )
    return f(a_resh, fit_flat, fit_labels)


def kernel(samples, train_data, train_labels):
    fit_data = train_data[::100]
    fit_labels = train_labels[::100]
    nkeys = fit_data.shape[0]
    devs = jax.devices()
    nd = max(d for d in range(1, len(devs) + 1) if nkeys % d == 0)
    if nd == 1:
        lab, _, _ = _run_shard(samples, fit_data, fit_labels)
        return lab

    kper = nkeys // nd

    def shard_fn(samples_r, fit_loc, lab_loc):
        lab, best, besti = _run_shard(samples_r, fit_loc, lab_loc)
        besti = besti + lax.axis_index("d").astype(jnp.int32) * kper
        return lab, best, besti

    mesh = jax.make_mesh((nd,), ("d",), devices=devs[:nd])
    samples_r = jax.reshard(samples, jax.NamedSharding(mesh, P()))
    fit_r = jax.reshard(fit_data, jax.NamedSharding(mesh, P("d")))
    labs_r = jax.reshard(fit_labels, jax.NamedSharding(mesh, P("d")))
    lab_s, best_s, besti_s = jax.shard_map(
        shard_fn,
        mesh=mesh,
        in_specs=(P(), P("d"), P("d")),
        out_specs=(P("d"), P("d"), P("d")),
        check_vma=False,
    )(samples_r, fit_r, labs_r)

    s = samples.shape[0]
    rep = jax.NamedSharding(mesh, P())
    lab_s = jax.reshard(lab_s, rep).reshape(nd, s)
    best_s = jax.reshard(best_s, rep).reshape(nd, s)
    besti_s = jax.reshard(besti_s, rep).reshape(nd, s)
    cur_b, cur_i, cur_l = best_s[0], besti_s[0], lab_s[0]
    for d in range(1, nd):
        better = (best_s[d] < cur_b) | ((best_s[d] == cur_b)
                                        & (besti_s[d] < cur_i))
        cur_b = jnp.where(better, best_s[d], cur_b)
        cur_i = jnp.where(better, besti_s[d], cur_i)
        cur_l = jnp.where(better, lab_s[d], cur_l)
    return cur_l


# speedup vs baseline: 4.4503x; 4.4503x over previous
import jax
import jax.numpy as jnp
from jax import lax
from jax.experimental import pallas as pl
from jax.experimental.pallas import tpu as pltpu
from jax.experimental.pallas import tpu_sc as plsc

_LANES = 16
_NW = 32
_WIN = 10
_BAND = 2 * _WIN + 1


def _dtw_knn_body(a_hbm, fit_hbm, lab_hbm, out_hbm, a_v, fit_v, lab_v, res_v):
    t = fit_hbm.shape[0] // lab_v.shape[0]
    nkeys = lab_v.shape[0]
    wid = lax.axis_index("s") * 2 + lax.axis_index("c")
    pltpu.sync_copy(a_hbm.at[wid], a_v)
    pltpu.sync_copy(fit_hbm, fit_v)
    pltpu.sync_copy(lab_hbm, lab_v)

    inf = jnp.full((_LANES,), jnp.inf, jnp.float32)

    def bcast_b(idx):
        return plsc.load_gather(fit_v, [idx])

    def key_body(jkey, carry):
        best, besti = carry
        kidx = jnp.full((_LANES,), jkey, jnp.int32)
        kbase = kidx * t
        klim = kbase + (t - 1)

        a0 = a_v[pl.ds(0, _LANES)]
        st = [inf] * _BAND
        run = jnp.abs(a0 - bcast_b(kbase))
        st[_WIN] = run
        for k in range(_WIN + 1, _BAND):
            run = run + jnp.abs(a0 - bcast_b(kbase + (k - _WIN)))
            st[k] = run

        def make_row(clamp_lo, clamp_hi):
            def row_body(i, carry_t):
                rb = carry_t[0]
                st = list(carry_t[1:])
                ai = a_v[pl.ds(i * _LANES, _LANES)]
                for k in range(_BAND - 1):
                    idx = rb + k if k else rb
                    if clamp_lo and k < _WIN:
                        cidx = jnp.maximum(idx, kbase)
                    elif clamp_hi and k > _WIN:
                        cidx = jnp.minimum(idx, klim)
                    else:
                        cidx = idx
                    c = jnp.abs(ai - bcast_b(cidx))
                    left = st[k - 1] if k >= 1 else inf
                    val = jnp.minimum(jnp.minimum(st[k], st[k + 1]), left) + c
                    if clamp_lo and k < _WIN:
                        val = jnp.where(idx >= kbase, val, inf)
                    elif clamp_hi and k > _WIN:
                        val = jnp.where(idx <= klim, val, inf)
                    st[k] = val
                st[_BAND - 1] = inf
                return (rb + 1,) + tuple(st)
            return row_body

        rb0 = kbase + (1 - _WIN)
        carry_t = (rb0,) + tuple(st)
        carry_t = lax.fori_loop(1, _WIN + 1, make_row(True, False), carry_t)
        carry_t = lax.fori_loop(_WIN + 1, t - _WIN + 1, make_row(False, False),
                                carry_t, unroll=2)
        carry_t = lax.fori_loop(t - _WIN + 1, t, make_row(False, True), carry_t)

        dist = carry_t[1 + _WIN]
        upd = dist < best
        best = jnp.where(upd, dist, best)
        besti = jnp.where(upd, kidx, besti)
        return best, besti

    best, besti = lax.fori_loop(
        0, nkeys,
        key_body,
        (inf, jnp.zeros((_LANES,), jnp.int32)),
    )
    res_v[...] = plsc.load_gather(lab_v, [besti])
    pltpu.sync_copy(res_v, out_hbm.at[pl.ds(wid * _LANES, _LANES)])


def kernel(samples, train_data, train_labels):
    fit_data = train_data[::100]
    fit_labels = train_labels[::100]
    s, t = samples.shape
    per_w = s // _NW
    a_resh = samples.reshape(_NW, per_w, t).transpose(0, 2, 1).reshape(_NW, -1)
    fit_flat = fit_data.reshape(-1)
    mesh = plsc.VectorSubcoreMesh(core_axis_name="c", subcore_axis_name="s")
    f = pl.kernel(
        _dtw_knn_body,
        out_type=jax.ShapeDtypeStruct((s,), jnp.int32),
        mesh=mesh,
        compiler_params=pltpu.CompilerParams(needs_layout_passes=False),
        scratch_types=[
            pltpu.VMEM((t * per_w,), jnp.float32),
            pltpu.VMEM((fit_flat.shape[0],), jnp.float32),
            pltpu.VMEM(fit_labels.shape, jnp.int32),
            pltpu.VMEM((per_w,), jnp.int32),
        ],
    )
    return f(a_resh, fit_flat, fit_labels)


# speedup vs baseline: 4.9672x; 1.1161x over previous
import jax
import jax.numpy as jnp
from jax import lax
from jax.experimental import pallas as pl
from jax.experimental.pallas import tpu as pltpu
from jax.experimental.pallas import tpu_sc as plsc

_LANES = 16
_NW = 32
_WIN = 10
_BAND = 2 * _WIN + 1


def _dtw_knn_body(a_hbm, fit_hbm, lab_hbm, out_hbm, a_v, fit_v, lab_v, res_v):
    t = fit_hbm.shape[0] // lab_v.shape[0]
    nkeys = lab_v.shape[0]
    wid = lax.axis_index("s") * 2 + lax.axis_index("c")
    pltpu.sync_copy(a_hbm.at[wid], a_v)
    pltpu.sync_copy(fit_hbm, fit_v)
    pltpu.sync_copy(lab_hbm, lab_v)

    inf = jnp.full((_LANES,), jnp.inf, jnp.float32)

    def bcast_b(idx):
        return plsc.load_gather(fit_v, [idx])

    def key_body(jkey, carry):
        best, besti = carry
        kidx = jnp.full((_LANES,), jkey, jnp.int32)
        kbase = kidx * t
        klim = kbase + (t - 1)

        a0 = a_v[pl.ds(0, _LANES)]
        st = [inf] * _BAND
        run = jnp.abs(a0 - bcast_b(kbase))
        st[_WIN] = run
        for k in range(_WIN + 1, _BAND):
            run = run + jnp.abs(a0 - bcast_b(kbase + (k - _WIN)))
            st[k] = run

        def min3(diag, top, left):
            terms = [x for x in (diag, top, left) if x is not inf]
            m = terms[0]
            for x in terms[1:]:
                m = jnp.minimum(m, x)
            return m

        def unrolled_row(i, st):
            ai = a_v[pl.ds(i * _LANES, _LANES)]
            new = [inf] * _BAND
            for k in range(_BAND - 1):
                j = i + k - _WIN
                if j < 0 or j > t - 1:
                    continue
                c = jnp.abs(ai - bcast_b(kbase + j if j else kbase))
                left = new[k - 1] if k >= 1 else inf
                new[k] = min3(st[k], st[k + 1], left) + c
            return new

        for i in range(1, _WIN + 1):
            st = unrolled_row(i, st)

        def row_body(i, carry_t):
            rb = carry_t[0]
            st = list(carry_t[1:]) + [inf]
            ai = a_v[pl.ds(i * _LANES, _LANES)]
            for k in range(_BAND - 1):
                c = jnp.abs(ai - bcast_b(rb + k if k else rb))
                left = st[k - 1] if k >= 1 else inf
                st[k] = min3(st[k], st[k + 1], left) + c
            return (rb + 1,) + tuple(st[:_BAND - 1])

        rb0 = kbase + 1
        carry_t = (rb0,) + tuple(st[:_BAND - 1])
        carry_t = lax.fori_loop(_WIN + 1, t - _WIN + 1, row_body, carry_t)
        st = list(carry_t[1:]) + [inf]

        for i in range(t - _WIN + 1, t):
            st = unrolled_row(i, st)

        dist = st[_WIN]
        upd = dist < best
        best = jnp.where(upd, dist, best)
        besti = jnp.where(upd, kidx, besti)
        return best, besti

    best, besti = lax.fori_loop(
        0, nkeys,
        key_body,
        (inf, jnp.zeros((_LANES,), jnp.int32)),
    )
    res_v[...] = plsc.load_gather(lab_v, [besti])
    pltpu.sync_copy(res_v, out_hbm.at[pl.ds(wid * _LANES, _LANES)])


def kernel(samples, train_data, train_labels):
    fit_data = train_data[::100]
    fit_labels = train_labels[::100]
    s, t = samples.shape
    per_w = s // _NW
    a_resh = samples.reshape(_NW, per_w, t).transpose(0, 2, 1).reshape(_NW, -1)
    fit_flat = fit_data.reshape(-1)
    mesh = plsc.VectorSubcoreMesh(core_axis_name="c", subcore_axis_name="s")
    f = pl.kernel(
        _dtw_knn_body,
        out_type=jax.ShapeDtypeStruct((s,), jnp.int32),
        mesh=mesh,
        compiler_params=pltpu.CompilerParams(needs_layout_passes=False),
        scratch_types=[
            pltpu.VMEM((t * per_w,), jnp.float32),
            pltpu.VMEM((fit_flat.shape[0],), jnp.float32),
            pltpu.VMEM(fit_labels.shape, jnp.int32),
            pltpu.VMEM((per_w,), jnp.int32),
        ],
    )
    return f(a_resh, fit_flat, fit_labels)
